# Initial kernel scaffold; baseline (speedup 1.0000x reference)
import jax
import jax.numpy as jnp
from jax.experimental import pallas as pl


def kernel(x):
    raise NotImplementedError("write your pallas kernel here")


# speedup vs baseline: 23.5536x; 23.5536x over previous
import jax
import jax.numpy as jnp
from jax.experimental import pallas as pl

_Z = 1.0
_NEWTON_ITERS = 12
_ROWS_PER_BLOCK = 8


def _body(x_ref, wp_ref, wc_ref):
    xb = x_ref[...]
    m0 = jnp.max(xb, axis=-1, keepdims=True)
    tau0 = m0 - _Z

    def step(_, tau):
        active = xb > tau
        cnt = jnp.sum(active.astype(jnp.float32), axis=-1, keepdims=True)
        ssum = jnp.sum(jnp.where(active, xb, 0.0), axis=-1, keepdims=True)
        return (ssum - _Z) / cnt

    tau = jax.lax.fori_loop(0, _NEWTON_ITERS, step, tau0)
    wp = jnp.maximum(xb - tau, 0.0)
    wp_ref[...] = wp
    wc_ref[...] = xb - wp


def kernel(x):
    b, n = x.shape
    grid = (b // _ROWS_PER_BLOCK,)
    spec = pl.BlockSpec((_ROWS_PER_BLOCK, n), lambda i: (i, 0))
    wp, wc = pl.pallas_call(
        _body,
        grid=grid,
        in_specs=[spec],
        out_specs=[spec, spec],
        out_shape=[
            jax.ShapeDtypeStruct(x.shape, x.dtype),
            jax.ShapeDtypeStruct(x.shape, x.dtype),
        ],
    )(x)
    return (wp, wc)
